# baseline (speedup 1.0000x reference)
import functools

import jax
import jax.numpy as jnp
from jax import lax
from jax.experimental import pallas as pl
from jax.experimental.pallas import tpu as pltpu
from jax.experimental.pallas import tpu_sc as plsc

S = 512
D = 768
R = 2 * S - 1
NW = 32
RPW = S // NW


def kernel(embedding, seq_len):
    del seq_len

    mesh = plsc.VectorSubcoreMesh(core_axis_name="c", subcore_axis_name="s")

    @functools.partial(
        pl.kernel,
        mesh=mesh,
        out_type=jax.ShapeDtypeStruct((S, S, D), jnp.float32),
        scratch_types=[
            pltpu.VMEM_SHARED((R, D), jnp.float32),
            pltpu.SemaphoreType.DMA,
            pltpu.SemaphoreType.DMA,
        ],
        compiler_params=pltpu.CompilerParams(use_tc_tiling_on_sc=False),
    )
    def sliding_copy(emb_hbm, out_hbm, table_sh, load_sem, store_sem):
        sid = lax.axis_index("s")
        cid = lax.axis_index("c")

        @pl.when(sid == 0)
        def _load_table():
            pltpu.async_copy(emb_hbm, table_sh, load_sem).wait()

        plsc.subcore_barrier()

        wid = sid * 2 + cid
        i0 = wid * RPW
        handles = []
        for r in range(RPW):
            i = i0 + r
            handles.append(
                pltpu.async_copy(
                    table_sh.at[pl.ds((S - 1) - i, S), :],
                    out_hbm.at[i],
                    store_sem,
                )
            )
        for h in handles:
            h.wait()

    return sliding_copy(embedding)


# speedup vs baseline: 1.1081x; 1.1081x over previous
import functools

import jax
import jax.numpy as jnp
from jax import lax
from jax.experimental import pallas as pl
from jax.experimental.pallas import tpu as pltpu
from jax.experimental.pallas import tpu_sc as plsc

S = 512
D = 768
NW = 32
RPW = S // NW
JB = 64
NJB = S // JB
CHUNK = JB + RPW - 1


def kernel(embedding, seq_len):
    del seq_len

    mesh = plsc.VectorSubcoreMesh(core_axis_name="c", subcore_axis_name="s")

    @functools.partial(
        pl.kernel,
        mesh=mesh,
        out_type=jax.ShapeDtypeStruct((S, S, D), jnp.float32),
        scratch_types=[
            pltpu.VMEM((2, CHUNK, D), jnp.float32),
            pltpu.SemaphoreType.DMA,
            pltpu.SemaphoreType.DMA,
            pltpu.SemaphoreType.DMA,
        ],
        compiler_params=pltpu.CompilerParams(use_tc_tiling_on_sc=False),
    )
    def sliding_copy(emb_hbm, out_hbm, chunk_v, load_sem, store_sem0, store_sem1):
        wid = lax.axis_index("s") * 2 + lax.axis_index("c")
        i0 = wid * RPW
        store_sems = (store_sem0, store_sem1)

        def start_load(jb, buf):
            base = jb * JB - i0 + (S - RPW)
            return pltpu.async_copy(
                emb_hbm.at[pl.ds(base, CHUNK), :], chunk_v.at[buf], load_sem
            )

        pending = {0: [], 1: []}
        load_h = start_load(0, 0)
        for jb in range(NJB):
            b = jb & 1
            load_h.wait()
            handles = []
            for r in range(RPW):
                handles.append(
                    pltpu.async_copy(
                        chunk_v.at[b, pl.ds(RPW - 1 - r, JB), :],
                        out_hbm.at[i0 + r, pl.ds(jb * JB, JB), :],
                        store_sems[b],
                    )
                )
            pending[b] = handles
            if jb + 1 < NJB:
                nb = 1 - b
                for h in pending[nb]:
                    h.wait()
                pending[nb] = []
                load_h = start_load(jb + 1, nb)
        for b in (0, 1):
            for h in pending[b]:
                h.wait()

    return sliding_copy(embedding)


# speedup vs baseline: 3.2525x; 2.9352x over previous
import functools

import jax
import jax.numpy as jnp
from jax import lax
from jax.experimental import pallas as pl
from jax.experimental.pallas import tpu as pltpu
from jax.experimental.pallas import tpu_sc as plsc

S = 512
D = 768
R = 2 * S - 1
NW = 32
RPW = S // NW
G = 8
NG = RPW // G
JB = 64
NJB = S // JB
CHUNK = 8 * (G - 1) + JB
RPAD = 1032


def kernel(embedding, seq_len):
    del seq_len

    emb8 = jnp.stack(
        [
            jnp.pad(embedding, ((k, RPAD - R - k), (0, 0)))
            for k in range(8)
        ]
    )

    mesh = plsc.VectorSubcoreMesh(core_axis_name="c", subcore_axis_name="s")

    @functools.partial(
        pl.kernel,
        mesh=mesh,
        out_type=jax.ShapeDtypeStruct((S, S, D), jnp.float32),
        scratch_types=[
            pltpu.VMEM((CHUNK, D), jnp.float32),
            pltpu.SemaphoreType.DMA,
            pltpu.SemaphoreType.DMA,
        ],
    )
    def sliding_copy(emb_hbm, out_hbm, chunk_v, load_sem, store_sem):
        wid = lax.axis_index("s") * 2 + lax.axis_index("c")
        res = lax.rem(wid, 8)
        q = wid // 8
        k = lax.rem(res + 1, 8)

        for h in range(NG):
            i_base = res + 8 * (RPW * q + G * h)
            for jb in range(NJB):
                j0 = jb * JB
                T = pl.multiple_of(
                    j0 - i_base + (S - 1 - 8 * (G - 1)) + k, 8
                )
                pltpu.async_copy(
                    emb_hbm.at[k, pl.ds(T, CHUNK), :], chunk_v, load_sem
                ).wait()
                handles = []
                for m in range(G):
                    handles.append(
                        pltpu.async_copy(
                            chunk_v.at[pl.ds(8 * (G - 1 - m), JB), :],
                            out_hbm.at[i_base + 8 * m, pl.ds(j0, JB), :],
                            store_sem,
                        )
                    )
                for hd in handles:
                    hd.wait()

    return sliding_copy(emb8)
